# Initial kernel scaffold; baseline (speedup 1.0000x reference)
import jax
import jax.numpy as jnp
from jax.experimental import pallas as pl


def kernel(x, edge_index, batch, W1, b1, W2, b2, W3, b3):
    raise NotImplementedError("write your pallas kernel here")


# speedup vs baseline: 12.7895x; 12.7895x over previous
import functools

import jax
import jax.numpy as jnp
from jax import lax
from jax.experimental import pallas as pl
from jax.experimental.pallas import tpu as pltpu
from jax.experimental.pallas import tpu_sc as plsc

N = 10000
E = 320000
G = 64
NC = 2
NS = 16
CHUNK = 128
EP = 327680
PAD = EP - E
EPT_FS = EP // NS
EPT_ES = EP // (NC * NS)
NP = 10240
RPT = NP // NS
ICH = 128
TRASH = NP - N
NDEG = 10752
BLK = 1000
GRID = N // BLK

_MESH = dict(core_axis_name="c", subcore_axis_name="s")


def _make_sc_scatter(edge_split):
    @functools.partial(
        pl.kernel,
        out_type=jax.ShapeDtypeStruct((2 * NP, 128), jnp.float32),
        mesh=plsc.VectorSubcoreMesh(**_MESH),
        scratch_types=[
            pltpu.VMEM((CHUNK,), jnp.int32),
            pltpu.VMEM((CHUNK,), jnp.int32),
            pltpu.VMEM((CHUNK, 128), jnp.float32),
            pltpu.VMEM_SHARED((NP, 128), jnp.float32),
            pltpu.SemaphoreType.DMA,
        ],
    )
    def scat(y_hbm, src_hbm, dst_hbm, out_hbm, sidx, didx, rows, acc, sem):
        c = lax.axis_index("c")
        s = lax.axis_index("s")
        ybase = 0 if edge_split else c * NP
        for k in range(RPT // ICH):
            pltpu.sync_copy(y_hbm.at[pl.ds(ybase + s * RPT + k * ICH, ICH)],
                            rows.at[pl.ds(0, ICH)])
            pltpu.sync_copy(rows.at[pl.ds(0, ICH)],
                            acc.at[pl.ds(s * RPT + k * ICH, ICH)])
        plsc.subcore_barrier()

        if edge_split:
            nch = EPT_ES // CHUNK
            ebase = (c * NS + s) * EPT_ES
            sbase = ebase
        else:
            nch = EPT_FS // CHUNK
            ebase = s * EPT_FS
            sbase = c * EP + ebase

        def body(j, carry):
            pltpu.sync_copy(src_hbm.at[pl.ds(sbase + j * CHUNK, CHUNK)], sidx)
            pltpu.sync_copy(dst_hbm.at[pl.ds(ebase + j * CHUNK, CHUNK)], didx)
            pltpu.async_copy(y_hbm.at[sidx], rows, sem).wait()
            pltpu.sync_copy(rows, acc.at[didx], add=True)
            return carry

        lax.fori_loop(0, nch, body, 0)
        plsc.subcore_barrier()
        for k in range(RPT // ICH):
            pltpu.sync_copy(acc.at[pl.ds(s * RPT + k * ICH, ICH)],
                            rows.at[pl.ds(0, ICH)])
            pltpu.sync_copy(rows.at[pl.ds(0, ICH)],
                            out_hbm.at[pl.ds(c * NP + s * RPT + k * ICH, ICH)])

    return scat


_sc_scatter_es = _make_sc_scatter(True)
_sc_scatter_fs = _make_sc_scatter(False)


def _make_sc_deg():
    zlen = NDEG // NS

    @functools.partial(
        pl.kernel,
        out_type=jax.ShapeDtypeStruct((NC * NDEG,), jnp.float32),
        mesh=plsc.VectorSubcoreMesh(**_MESH),
        scratch_types=[
            pltpu.VMEM((CHUNK,), jnp.int32),
            pltpu.VMEM((CHUNK,), jnp.float32),
            pltpu.VMEM((zlen,), jnp.float32),
            pltpu.VMEM_SHARED((NDEG,), jnp.float32),
        ],
    )
    def degk(dst_hbm, out_hbm, didx, ones_v, zbuf, acc):
        c = lax.axis_index("c")
        s = lax.axis_index("s")
        for i in range(CHUNK // 16):
            ones_v[pl.ds(16 * i, 16)] = jnp.ones((16,), jnp.float32)
        for i in range(zlen // 16):
            zbuf[pl.ds(16 * i, 16)] = jnp.zeros((16,), jnp.float32)
        pltpu.sync_copy(zbuf, acc.at[pl.ds(s * zlen, zlen)])
        plsc.subcore_barrier()

        def body(j, carry):
            e0 = (c * NS + s) * EPT_ES + j * CHUNK
            pltpu.sync_copy(dst_hbm.at[pl.ds(e0, CHUNK)], didx)
            pltpu.sync_copy(ones_v, acc.at[didx], add=True)
            return carry

        lax.fori_loop(0, EPT_ES // CHUNK, body, 0)
        plsc.subcore_barrier()
        pltpu.sync_copy(acc.at[pl.ds(s * zlen, zlen)], zbuf)
        pltpu.sync_copy(zbuf, out_hbm.at[pl.ds(c * NDEG + s * zlen, zlen)])

    return degk


_sc_deg = _make_sc_deg()


def _tc1_body(deg_ref, x_ref, dinv_ref, y0_ref):
    d = deg_ref[0] + deg_ref[1]
    dinv = lax.rsqrt(d + 1.0)
    dinv_ref[...] = dinv
    y0_ref[...] = x_ref[...] * dinv


def _tc1(degs, x):
    return pl.pallas_call(
        _tc1_body,
        grid=(GRID,),
        in_specs=[
            pl.BlockSpec((2, BLK, 1), lambda i: (0, i, 0)),
            pl.BlockSpec((BLK, 128), lambda i: (i, 0)),
        ],
        out_specs=[
            pl.BlockSpec((BLK, 1), lambda i: (i, 0)),
            pl.BlockSpec((BLK, 128), lambda i: (i, 0)),
        ],
        out_shape=[
            jax.ShapeDtypeStruct((N, 1), jnp.float32),
            jax.ShapeDtypeStruct((NP, 128), jnp.float32),
        ],
    )(degs, x)


def _tc2_body(z_ref, y0_ref, dinv_ref, w1_ref, b1_ref, y1_ref):
    dinv = dinv_ref[...]
    a = (z_ref[0] + z_ref[1] - y0_ref[...]) * dinv
    h = jnp.dot(a, w1_ref[...], preferred_element_type=jnp.float32)
    h = jnp.maximum(h + b1_ref[...], 0.0)
    y = h * dinv
    y1_ref[0] = y[:, :128]
    y1_ref[1] = y[:, 128:]


def _tc2(z1, y0, dinv, W1, b1):
    return pl.pallas_call(
        _tc2_body,
        grid=(GRID,),
        in_specs=[
            pl.BlockSpec((2, BLK, 128), lambda i: (0, i, 0)),
            pl.BlockSpec((BLK, 128), lambda i: (i, 0)),
            pl.BlockSpec((BLK, 1), lambda i: (i, 0)),
            pl.BlockSpec((128, 256), lambda i: (0, 0)),
            pl.BlockSpec((1, 256), lambda i: (0, 0)),
        ],
        out_specs=pl.BlockSpec((2, BLK, 128), lambda i: (0, i, 0)),
        out_shape=jax.ShapeDtypeStruct((2, NP, 128), jnp.float32),
    )(z1, y0, dinv, W1, b1)


def _tc3_body(z_ref, dinv_ref, w2_ref, b2_ref, w3_ref, y3_ref):
    dinv = dinv_ref[...]
    a = jnp.concatenate([z_ref[0], z_ref[1]], axis=1) * dinv
    h = jnp.dot(a, w2_ref[...], preferred_element_type=jnp.float32)
    h = jnp.maximum(h + b2_ref[...], 0.0)
    y3_ref[...] = jnp.dot(h, w3_ref[...],
                          preferred_element_type=jnp.float32) * dinv


def _tc3(z2, dinv, W2, b2, W3):
    return pl.pallas_call(
        _tc3_body,
        grid=(GRID,),
        in_specs=[
            pl.BlockSpec((2, BLK, 128), lambda i: (0, i, 0)),
            pl.BlockSpec((BLK, 1), lambda i: (i, 0)),
            pl.BlockSpec((256, 256), lambda i: (0, 0)),
            pl.BlockSpec((1, 256), lambda i: (0, 0)),
            pl.BlockSpec((256, 128), lambda i: (0, 0)),
        ],
        out_specs=pl.BlockSpec((BLK, 128), lambda i: (i, 0)),
        out_shape=jax.ShapeDtypeStruct((NP, 128), jnp.float32),
    )(z2, dinv, W2, b2, W3)


def _tc4_body(z_ref, y3_ref, dinv_ref, b3_ref, batch_ref, node_ref, graph_ref,
              pooled_acc, counts_acc):
    i = pl.program_id(0)
    node = ((z_ref[0] + z_ref[1] - y3_ref[...]) * dinv_ref[...]
            + b3_ref[...])
    node_ref[...] = node
    onehot = (batch_ref[...] ==
              lax.broadcasted_iota(jnp.int32, (BLK, G), 1)).astype(jnp.float32)
    pooled = lax.dot_general(onehot, node, (((0,), (0,)), ((), ())),
                             preferred_element_type=jnp.float32)
    cnt = lax.dot_general(onehot, jnp.ones((BLK, 1), jnp.float32),
                          (((0,), (0,)), ((), ())),
                          preferred_element_type=jnp.float32)

    @pl.when(i == 0)
    def _():
        pooled_acc[...] = jnp.zeros_like(pooled_acc)
        counts_acc[...] = jnp.zeros_like(counts_acc)

    pooled_acc[...] += pooled
    counts_acc[...] += cnt

    @pl.when(i == pl.num_programs(0) - 1)
    def _():
        graph_ref[...] = pooled_acc[...] / jnp.maximum(counts_acc[...], 1.0)


def _tc4(z3, y3, dinv, b3, batch2):
    return pl.pallas_call(
        _tc4_body,
        grid=(GRID,),
        in_specs=[
            pl.BlockSpec((2, BLK, 128), lambda i: (0, i, 0)),
            pl.BlockSpec((BLK, 128), lambda i: (i, 0)),
            pl.BlockSpec((BLK, 1), lambda i: (i, 0)),
            pl.BlockSpec((1, 128), lambda i: (0, 0)),
            pl.BlockSpec((BLK, 1), lambda i: (i, 0)),
        ],
        out_specs=[
            pl.BlockSpec((BLK, 128), lambda i: (i, 0)),
            pl.BlockSpec((G, 128), lambda i: (0, 0)),
        ],
        out_shape=[
            jax.ShapeDtypeStruct((N, 128), jnp.float32),
            jax.ShapeDtypeStruct((G, 128), jnp.float32),
        ],
        scratch_shapes=[
            pltpu.VMEM((G, 128), jnp.float32),
            pltpu.VMEM((G, 1), jnp.float32),
        ],
    )(z3, y3, dinv, b3, batch2)


def kernel(x, edge_index, batch, W1, b1, W2, b2, W3, b3):
    src = edge_index[0].astype(jnp.int32)
    dst = edge_index[1].astype(jnp.int32)
    pad_i = jnp.arange(PAD, dtype=jnp.int32)
    src_p = jnp.concatenate([src, pad_i % N])
    dst_p = jnp.concatenate([dst, N + (pad_i % TRASH)])
    src_fs = jnp.concatenate([src_p, src_p + NP])

    deg_flat = _sc_deg(dst_p)
    degs = deg_flat.reshape(NC, NDEG)[:, :N].reshape(2, N, 1)

    dinv, y0 = _tc1(degs, x)
    z1 = _sc_scatter_es(y0, src_p, dst_p).reshape(2, NP, 128)
    y1 = _tc2(z1, y0, dinv, W1, b1.reshape(1, 256))
    z2 = _sc_scatter_fs(y1.reshape(2 * NP, 128), src_fs, dst_p)
    y3 = _tc3(z2.reshape(2, NP, 128), dinv, W2, b2.reshape(1, 256), W3)
    z3 = _sc_scatter_es(y3, src_p, dst_p).reshape(2, NP, 128)
    node_repr, graph_repr = _tc4(z3, y3, dinv, b3.reshape(1, 128),
                                 batch.astype(jnp.int32).reshape(N, 1))
    return (node_repr, graph_repr)


# speedup vs baseline: 16.7502x; 1.3097x over previous
import functools

import jax
import jax.numpy as jnp
from jax import lax
from jax.experimental import pallas as pl
from jax.experimental.pallas import tpu as pltpu
from jax.experimental.pallas import tpu_sc as plsc

N = 10000
E = 320000
G = 64
NC = 2
NS = 16
CHUNK = 256
EP = 327680
PAD = EP - E
EPT_FS = EP // NS
EPT_ES = EP // (NC * NS)
NP = 10240
RPT = NP // NS
ICH = 128
TRASH = NP - N
NDEG = 10752
BLK = 1000
GRID = N // BLK

_MESH = dict(core_axis_name="c", subcore_axis_name="s")


def _make_sc_scatter(edge_split):
    @functools.partial(
        pl.kernel,
        out_type=jax.ShapeDtypeStruct((2 * NP, 128), jnp.float32),
        mesh=plsc.VectorSubcoreMesh(**_MESH),
        scratch_types=[
            pltpu.VMEM((CHUNK,), jnp.int32),
            pltpu.VMEM((CHUNK,), jnp.int32),
            pltpu.VMEM((CHUNK, 128), jnp.float32),
            pltpu.VMEM_SHARED((NP, 128), jnp.float32),
            pltpu.SemaphoreType.DMA,
        ],
    )
    def scat(y_hbm, src_hbm, dst_hbm, out_hbm, sidx, didx, rows, acc, sem):
        c = lax.axis_index("c")
        s = lax.axis_index("s")
        ybase = 0 if edge_split else c * NP
        for k in range(RPT // ICH):
            pltpu.sync_copy(y_hbm.at[pl.ds(ybase + s * RPT + k * ICH, ICH)],
                            rows.at[pl.ds(0, ICH)])
            pltpu.sync_copy(rows.at[pl.ds(0, ICH)],
                            acc.at[pl.ds(s * RPT + k * ICH, ICH)])
        plsc.subcore_barrier()

        if edge_split:
            nch = EPT_ES // CHUNK
            ebase = (c * NS + s) * EPT_ES
            sbase = ebase
        else:
            nch = EPT_FS // CHUNK
            ebase = s * EPT_FS
            sbase = c * EP + ebase

        def body(j, carry):
            pltpu.sync_copy(src_hbm.at[pl.ds(sbase + j * CHUNK, CHUNK)], sidx)
            pltpu.sync_copy(dst_hbm.at[pl.ds(ebase + j * CHUNK, CHUNK)], didx)
            pltpu.async_copy(y_hbm.at[sidx], rows, sem).wait()
            pltpu.sync_copy(rows, acc.at[didx], add=True)
            return carry

        lax.fori_loop(0, nch, body, 0)
        plsc.subcore_barrier()
        for k in range(RPT // ICH):
            pltpu.sync_copy(acc.at[pl.ds(s * RPT + k * ICH, ICH)],
                            rows.at[pl.ds(0, ICH)])
            pltpu.sync_copy(rows.at[pl.ds(0, ICH)],
                            out_hbm.at[pl.ds(c * NP + s * RPT + k * ICH, ICH)])

    return scat


_sc_scatter_es = _make_sc_scatter(True)
_sc_scatter_fs = _make_sc_scatter(False)


def _make_sc_deg():
    zlen = NDEG // NS

    @functools.partial(
        pl.kernel,
        out_type=jax.ShapeDtypeStruct((NC * NDEG,), jnp.float32),
        mesh=plsc.VectorSubcoreMesh(**_MESH),
        scratch_types=[
            pltpu.VMEM((CHUNK,), jnp.int32),
            pltpu.VMEM((CHUNK,), jnp.float32),
            pltpu.VMEM((zlen,), jnp.float32),
            pltpu.VMEM_SHARED((NDEG,), jnp.float32),
        ],
    )
    def degk(dst_hbm, out_hbm, didx, ones_v, zbuf, acc):
        c = lax.axis_index("c")
        s = lax.axis_index("s")
        for i in range(CHUNK // 16):
            ones_v[pl.ds(16 * i, 16)] = jnp.ones((16,), jnp.float32)
        for i in range(zlen // 16):
            zbuf[pl.ds(16 * i, 16)] = jnp.zeros((16,), jnp.float32)
        pltpu.sync_copy(zbuf, acc.at[pl.ds(s * zlen, zlen)])
        plsc.subcore_barrier()

        def body(j, carry):
            e0 = (c * NS + s) * EPT_ES + j * CHUNK
            pltpu.sync_copy(dst_hbm.at[pl.ds(e0, CHUNK)], didx)
            pltpu.sync_copy(ones_v, acc.at[didx], add=True)
            return carry

        lax.fori_loop(0, EPT_ES // CHUNK, body, 0)
        plsc.subcore_barrier()
        pltpu.sync_copy(acc.at[pl.ds(s * zlen, zlen)], zbuf)
        pltpu.sync_copy(zbuf, out_hbm.at[pl.ds(c * NDEG + s * zlen, zlen)])

    return degk


_sc_deg = _make_sc_deg()


def _tc1_body(deg_ref, x_ref, dinv_ref, y0_ref):
    d = deg_ref[0] + deg_ref[1]
    dinv = lax.rsqrt(d + 1.0)
    dinv_ref[...] = dinv
    y0_ref[...] = x_ref[...] * dinv


def _tc1(degs, x):
    return pl.pallas_call(
        _tc1_body,
        grid=(GRID,),
        in_specs=[
            pl.BlockSpec((2, BLK, 1), lambda i: (0, i, 0)),
            pl.BlockSpec((BLK, 128), lambda i: (i, 0)),
        ],
        out_specs=[
            pl.BlockSpec((BLK, 1), lambda i: (i, 0)),
            pl.BlockSpec((BLK, 128), lambda i: (i, 0)),
        ],
        out_shape=[
            jax.ShapeDtypeStruct((N, 1), jnp.float32),
            jax.ShapeDtypeStruct((NP, 128), jnp.float32),
        ],
    )(degs, x)


def _tc2_body(z_ref, y0_ref, dinv_ref, w1_ref, b1_ref, y1_ref):
    dinv = dinv_ref[...]
    a = (z_ref[0] + z_ref[1] - y0_ref[...]) * dinv
    h = jnp.dot(a, w1_ref[...], preferred_element_type=jnp.float32)
    h = jnp.maximum(h + b1_ref[...], 0.0)
    y = h * dinv
    y1_ref[0] = y[:, :128]
    y1_ref[1] = y[:, 128:]


def _tc2(z1, y0, dinv, W1, b1):
    return pl.pallas_call(
        _tc2_body,
        grid=(GRID,),
        in_specs=[
            pl.BlockSpec((2, BLK, 128), lambda i: (0, i, 0)),
            pl.BlockSpec((BLK, 128), lambda i: (i, 0)),
            pl.BlockSpec((BLK, 1), lambda i: (i, 0)),
            pl.BlockSpec((128, 256), lambda i: (0, 0)),
            pl.BlockSpec((1, 256), lambda i: (0, 0)),
        ],
        out_specs=pl.BlockSpec((2, BLK, 128), lambda i: (0, i, 0)),
        out_shape=jax.ShapeDtypeStruct((2, NP, 128), jnp.float32),
    )(z1, y0, dinv, W1, b1)


def _tc3_body(z_ref, dinv_ref, w2_ref, b2_ref, w3_ref, y3_ref):
    dinv = dinv_ref[...]
    a = jnp.concatenate([z_ref[0], z_ref[1]], axis=1) * dinv
    h = jnp.dot(a, w2_ref[...], preferred_element_type=jnp.float32)
    h = jnp.maximum(h + b2_ref[...], 0.0)
    y3_ref[...] = jnp.dot(h, w3_ref[...],
                          preferred_element_type=jnp.float32) * dinv


def _tc3(z2, dinv, W2, b2, W3):
    return pl.pallas_call(
        _tc3_body,
        grid=(GRID,),
        in_specs=[
            pl.BlockSpec((2, BLK, 128), lambda i: (0, i, 0)),
            pl.BlockSpec((BLK, 1), lambda i: (i, 0)),
            pl.BlockSpec((256, 256), lambda i: (0, 0)),
            pl.BlockSpec((1, 256), lambda i: (0, 0)),
            pl.BlockSpec((256, 128), lambda i: (0, 0)),
        ],
        out_specs=pl.BlockSpec((BLK, 128), lambda i: (i, 0)),
        out_shape=jax.ShapeDtypeStruct((NP, 128), jnp.float32),
    )(z2, dinv, W2, b2, W3)


def _tc4_body(z_ref, y3_ref, dinv_ref, b3_ref, batch_ref, node_ref, graph_ref,
              pooled_acc, counts_acc):
    i = pl.program_id(0)
    node = ((z_ref[0] + z_ref[1] - y3_ref[...]) * dinv_ref[...]
            + b3_ref[...])
    node_ref[...] = node
    onehot = (batch_ref[...] ==
              lax.broadcasted_iota(jnp.int32, (BLK, G), 1)).astype(jnp.float32)
    pooled = lax.dot_general(onehot, node, (((0,), (0,)), ((), ())),
                             preferred_element_type=jnp.float32)
    cnt = lax.dot_general(onehot, jnp.ones((BLK, 1), jnp.float32),
                          (((0,), (0,)), ((), ())),
                          preferred_element_type=jnp.float32)

    @pl.when(i == 0)
    def _():
        pooled_acc[...] = jnp.zeros_like(pooled_acc)
        counts_acc[...] = jnp.zeros_like(counts_acc)

    pooled_acc[...] += pooled
    counts_acc[...] += cnt

    @pl.when(i == pl.num_programs(0) - 1)
    def _():
        graph_ref[...] = pooled_acc[...] / jnp.maximum(counts_acc[...], 1.0)


def _tc4(z3, y3, dinv, b3, batch2):
    return pl.pallas_call(
        _tc4_body,
        grid=(GRID,),
        in_specs=[
            pl.BlockSpec((2, BLK, 128), lambda i: (0, i, 0)),
            pl.BlockSpec((BLK, 128), lambda i: (i, 0)),
            pl.BlockSpec((BLK, 1), lambda i: (i, 0)),
            pl.BlockSpec((1, 128), lambda i: (0, 0)),
            pl.BlockSpec((BLK, 1), lambda i: (i, 0)),
        ],
        out_specs=[
            pl.BlockSpec((BLK, 128), lambda i: (i, 0)),
            pl.BlockSpec((G, 128), lambda i: (0, 0)),
        ],
        out_shape=[
            jax.ShapeDtypeStruct((N, 128), jnp.float32),
            jax.ShapeDtypeStruct((G, 128), jnp.float32),
        ],
        scratch_shapes=[
            pltpu.VMEM((G, 128), jnp.float32),
            pltpu.VMEM((G, 1), jnp.float32),
        ],
    )(z3, y3, dinv, b3, batch2)


def kernel(x, edge_index, batch, W1, b1, W2, b2, W3, b3):
    src = edge_index[0].astype(jnp.int32)
    dst = edge_index[1].astype(jnp.int32)
    pad_i = jnp.arange(PAD, dtype=jnp.int32)
    src_p = jnp.concatenate([src, pad_i % N])
    dst_p = jnp.concatenate([dst, N + (pad_i % TRASH)])
    src_fs = jnp.concatenate([src_p, src_p + NP])

    deg_flat = _sc_deg(dst_p)
    degs = deg_flat.reshape(NC, NDEG)[:, :N].reshape(2, N, 1)

    dinv, y0 = _tc1(degs, x)
    z1 = _sc_scatter_es(y0, src_p, dst_p).reshape(2, NP, 128)
    y1 = _tc2(z1, y0, dinv, W1, b1.reshape(1, 256))
    z2 = _sc_scatter_fs(y1.reshape(2 * NP, 128), src_fs, dst_p)
    y3 = _tc3(z2.reshape(2, NP, 128), dinv, W2, b2.reshape(1, 256), W3)
    z3 = _sc_scatter_es(y3, src_p, dst_p).reshape(2, NP, 128)
    node_repr, graph_repr = _tc4(z3, y3, dinv, b3.reshape(1, 128),
                                 batch.astype(jnp.int32).reshape(N, 1))
    return (node_repr, graph_repr)


# speedup vs baseline: 20.9678x; 1.2518x over previous
import functools

import jax
import jax.numpy as jnp
from jax import lax
from jax.experimental import pallas as pl
from jax.experimental.pallas import tpu as pltpu
from jax.experimental.pallas import tpu_sc as plsc

N = 10000
E = 320000
G = 64
NC = 2
NS = 16
CHUNK = 128
IB = 2048
EP = 327680
PAD = EP - E
EPT_FS = EP // NS
EPT_ES = EP // (NC * NS)
NP = 10240
RPT = NP // NS
ICH = 128
TRASH = NP - N
NDEG = 10752
BLK = 1000
GRID = N // BLK

_MESH = dict(core_axis_name="c", subcore_axis_name="s")


def _make_sc_scatter(edge_split):
    @functools.partial(
        pl.kernel,
        out_type=jax.ShapeDtypeStruct((2 * NP, 128), jnp.float32),
        mesh=plsc.VectorSubcoreMesh(**_MESH),
        scratch_types=[
            pltpu.VMEM((IB,), jnp.int32),
            pltpu.VMEM((IB,), jnp.int32),
            pltpu.VMEM((CHUNK, 128), jnp.float32),
            pltpu.VMEM((CHUNK, 128), jnp.float32),
            pltpu.VMEM_SHARED((NP, 128), jnp.float32),
            pltpu.SemaphoreType.DMA,
            pltpu.SemaphoreType.DMA,
            pltpu.SemaphoreType.DMA,
            pltpu.SemaphoreType.DMA,
        ],
    )
    def scat(y_hbm, src_hbm, dst_hbm, out_hbm, sidx, didx,
             rows0, rows1, acc, gsem0, gsem1, ssem0, ssem1):
        c = lax.axis_index("c")
        s = lax.axis_index("s")
        ybase = 0 if edge_split else c * NP
        for k in range(RPT // ICH):
            pltpu.sync_copy(y_hbm.at[pl.ds(ybase + s * RPT + k * ICH, ICH)],
                            rows0.at[pl.ds(0, ICH)])
            pltpu.sync_copy(rows0.at[pl.ds(0, ICH)],
                            acc.at[pl.ds(s * RPT + k * ICH, ICH)])
        plsc.subcore_barrier()

        if edge_split:
            nblk = EPT_ES // IB
            ebase = (c * NS + s) * EPT_ES
            sbase = ebase
        else:
            nblk = EPT_FS // IB
            ebase = s * EPT_FS
            sbase = c * EP + ebase

        rows = (rows0, rows1)
        gsem = (gsem0, gsem1)
        ssem = (ssem0, ssem1)
        nk = IB // CHUNK

        def blk(t, carry):
            pltpu.sync_copy(src_hbm.at[pl.ds(sbase + t * IB, IB)], sidx)
            pltpu.sync_copy(dst_hbm.at[pl.ds(ebase + t * IB, IB)], didx)
            pltpu.async_copy(
                y_hbm.at[sidx.at[pl.ds(0, CHUNK)]], rows[0], gsem[0])
            for k in range(nk):
                b = k % 2
                o = 1 - b
                gk = y_hbm.at[sidx.at[pl.ds(k * CHUNK, CHUNK)]]
                pltpu.make_async_copy(gk, rows[b], gsem[b]).wait()
                pltpu.async_copy(rows[b],
                                 acc.at[didx.at[pl.ds(k * CHUNK, CHUNK)]],
                                 ssem[b], add=True)
                if k >= 1:
                    pltpu.make_async_copy(
                        rows[o], acc.at[didx.at[pl.ds((k - 1) * CHUNK, CHUNK)]],
                        ssem[o]).wait()
                if k + 1 < nk:
                    pltpu.async_copy(
                        y_hbm.at[sidx.at[pl.ds((k + 1) * CHUNK, CHUNK)]],
                        rows[o], gsem[o])
            last = nk - 1
            pltpu.make_async_copy(
                rows[last % 2], acc.at[didx.at[pl.ds(last * CHUNK, CHUNK)]],
                ssem[last % 2]).wait()
            return carry

        lax.fori_loop(0, nblk, blk, 0)
        plsc.subcore_barrier()
        for k in range(RPT // ICH):
            pltpu.sync_copy(acc.at[pl.ds(s * RPT + k * ICH, ICH)],
                            rows0.at[pl.ds(0, ICH)])
            pltpu.sync_copy(rows0.at[pl.ds(0, ICH)],
                            out_hbm.at[pl.ds(c * NP + s * RPT + k * ICH, ICH)])

    return scat


_sc_scatter_es = _make_sc_scatter(True)
_sc_scatter_fs = _make_sc_scatter(False)


def _make_sc_deg():
    zlen = NDEG // NS

    @functools.partial(
        pl.kernel,
        out_type=jax.ShapeDtypeStruct((NC * NDEG,), jnp.float32),
        mesh=plsc.VectorSubcoreMesh(**_MESH),
        scratch_types=[
            pltpu.VMEM((CHUNK,), jnp.int32),
            pltpu.VMEM((CHUNK,), jnp.float32),
            pltpu.VMEM((zlen,), jnp.float32),
            pltpu.VMEM_SHARED((NDEG,), jnp.float32),
        ],
    )
    def degk(dst_hbm, out_hbm, didx, ones_v, zbuf, acc):
        c = lax.axis_index("c")
        s = lax.axis_index("s")
        for i in range(CHUNK // 16):
            ones_v[pl.ds(16 * i, 16)] = jnp.ones((16,), jnp.float32)
        for i in range(zlen // 16):
            zbuf[pl.ds(16 * i, 16)] = jnp.zeros((16,), jnp.float32)
        pltpu.sync_copy(zbuf, acc.at[pl.ds(s * zlen, zlen)])
        plsc.subcore_barrier()

        def body(j, carry):
            e0 = (c * NS + s) * EPT_ES + j * CHUNK
            pltpu.sync_copy(dst_hbm.at[pl.ds(e0, CHUNK)], didx)
            pltpu.sync_copy(ones_v, acc.at[didx], add=True)
            return carry

        lax.fori_loop(0, EPT_ES // CHUNK, body, 0)
        plsc.subcore_barrier()
        pltpu.sync_copy(acc.at[pl.ds(s * zlen, zlen)], zbuf)
        pltpu.sync_copy(zbuf, out_hbm.at[pl.ds(c * NDEG + s * zlen, zlen)])

    return degk


_sc_deg = _make_sc_deg()


def _tc1_body(deg_ref, x_ref, dinv_ref, y0_ref):
    d = deg_ref[0] + deg_ref[1]
    dinv = lax.rsqrt(d + 1.0)
    dinv_ref[...] = dinv
    y0_ref[...] = x_ref[...] * dinv


def _tc1(degs, x):
    return pl.pallas_call(
        _tc1_body,
        grid=(GRID,),
        in_specs=[
            pl.BlockSpec((2, BLK, 1), lambda i: (0, i, 0)),
            pl.BlockSpec((BLK, 128), lambda i: (i, 0)),
        ],
        out_specs=[
            pl.BlockSpec((BLK, 1), lambda i: (i, 0)),
            pl.BlockSpec((BLK, 128), lambda i: (i, 0)),
        ],
        out_shape=[
            jax.ShapeDtypeStruct((N, 1), jnp.float32),
            jax.ShapeDtypeStruct((NP, 128), jnp.float32),
        ],
    )(degs, x)


def _tc2_body(z_ref, y0_ref, dinv_ref, w1_ref, b1_ref, y1_ref):
    dinv = dinv_ref[...]
    a = (z_ref[0] + z_ref[1] - y0_ref[...]) * dinv
    h = jnp.dot(a, w1_ref[...], preferred_element_type=jnp.float32)
    h = jnp.maximum(h + b1_ref[...], 0.0)
    y = h * dinv
    y1_ref[0] = y[:, :128]
    y1_ref[1] = y[:, 128:]


def _tc2(z1, y0, dinv, W1, b1):
    return pl.pallas_call(
        _tc2_body,
        grid=(GRID,),
        in_specs=[
            pl.BlockSpec((2, BLK, 128), lambda i: (0, i, 0)),
            pl.BlockSpec((BLK, 128), lambda i: (i, 0)),
            pl.BlockSpec((BLK, 1), lambda i: (i, 0)),
            pl.BlockSpec((128, 256), lambda i: (0, 0)),
            pl.BlockSpec((1, 256), lambda i: (0, 0)),
        ],
        out_specs=pl.BlockSpec((2, BLK, 128), lambda i: (0, i, 0)),
        out_shape=jax.ShapeDtypeStruct((2, NP, 128), jnp.float32),
    )(z1, y0, dinv, W1, b1)


def _tc3_body(z_ref, dinv_ref, w2_ref, b2_ref, w3_ref, y3_ref):
    dinv = dinv_ref[...]
    a = jnp.concatenate([z_ref[0], z_ref[1]], axis=1) * dinv
    h = jnp.dot(a, w2_ref[...], preferred_element_type=jnp.float32)
    h = jnp.maximum(h + b2_ref[...], 0.0)
    y3_ref[...] = jnp.dot(h, w3_ref[...],
                          preferred_element_type=jnp.float32) * dinv


def _tc3(z2, dinv, W2, b2, W3):
    return pl.pallas_call(
        _tc3_body,
        grid=(GRID,),
        in_specs=[
            pl.BlockSpec((2, BLK, 128), lambda i: (0, i, 0)),
            pl.BlockSpec((BLK, 1), lambda i: (i, 0)),
            pl.BlockSpec((256, 256), lambda i: (0, 0)),
            pl.BlockSpec((1, 256), lambda i: (0, 0)),
            pl.BlockSpec((256, 128), lambda i: (0, 0)),
        ],
        out_specs=pl.BlockSpec((BLK, 128), lambda i: (i, 0)),
        out_shape=jax.ShapeDtypeStruct((NP, 128), jnp.float32),
    )(z2, dinv, W2, b2, W3)


def _tc4_body(z_ref, y3_ref, dinv_ref, b3_ref, batch_ref, node_ref, graph_ref,
              pooled_acc, counts_acc):
    i = pl.program_id(0)
    node = ((z_ref[0] + z_ref[1] - y3_ref[...]) * dinv_ref[...]
            + b3_ref[...])
    node_ref[...] = node
    onehot = (batch_ref[...] ==
              lax.broadcasted_iota(jnp.int32, (BLK, G), 1)).astype(jnp.float32)
    pooled = lax.dot_general(onehot, node, (((0,), (0,)), ((), ())),
                             preferred_element_type=jnp.float32)
    cnt = lax.dot_general(onehot, jnp.ones((BLK, 1), jnp.float32),
                          (((0,), (0,)), ((), ())),
                          preferred_element_type=jnp.float32)

    @pl.when(i == 0)
    def _():
        pooled_acc[...] = jnp.zeros_like(pooled_acc)
        counts_acc[...] = jnp.zeros_like(counts_acc)

    pooled_acc[...] += pooled
    counts_acc[...] += cnt

    @pl.when(i == pl.num_programs(0) - 1)
    def _():
        graph_ref[...] = pooled_acc[...] / jnp.maximum(counts_acc[...], 1.0)


def _tc4(z3, y3, dinv, b3, batch2):
    return pl.pallas_call(
        _tc4_body,
        grid=(GRID,),
        in_specs=[
            pl.BlockSpec((2, BLK, 128), lambda i: (0, i, 0)),
            pl.BlockSpec((BLK, 128), lambda i: (i, 0)),
            pl.BlockSpec((BLK, 1), lambda i: (i, 0)),
            pl.BlockSpec((1, 128), lambda i: (0, 0)),
            pl.BlockSpec((BLK, 1), lambda i: (i, 0)),
        ],
        out_specs=[
            pl.BlockSpec((BLK, 128), lambda i: (i, 0)),
            pl.BlockSpec((G, 128), lambda i: (0, 0)),
        ],
        out_shape=[
            jax.ShapeDtypeStruct((N, 128), jnp.float32),
            jax.ShapeDtypeStruct((G, 128), jnp.float32),
        ],
        scratch_shapes=[
            pltpu.VMEM((G, 128), jnp.float32),
            pltpu.VMEM((G, 1), jnp.float32),
        ],
    )(z3, y3, dinv, b3, batch2)


def kernel(x, edge_index, batch, W1, b1, W2, b2, W3, b3):
    src = edge_index[0].astype(jnp.int32)
    dst = edge_index[1].astype(jnp.int32)
    pad_i = jnp.arange(PAD, dtype=jnp.int32)
    src_p = jnp.concatenate([src, pad_i % N])
    dst_p = jnp.concatenate([dst, N + (pad_i % TRASH)])
    src_fs = jnp.concatenate([src_p, src_p + NP])

    deg_flat = _sc_deg(dst_p)
    degs = deg_flat.reshape(NC, NDEG)[:, :N].reshape(2, N, 1)

    dinv, y0 = _tc1(degs, x)
    z1 = _sc_scatter_es(y0, src_p, dst_p).reshape(2, NP, 128)
    y1 = _tc2(z1, y0, dinv, W1, b1.reshape(1, 256))
    z2 = _sc_scatter_fs(y1.reshape(2 * NP, 128), src_fs, dst_p)
    y3 = _tc3(z2.reshape(2, NP, 128), dinv, W2, b2.reshape(1, 256), W3)
    z3 = _sc_scatter_es(y3, src_p, dst_p).reshape(2, NP, 128)
    node_repr, graph_repr = _tc4(z3, y3, dinv, b3.reshape(1, 128),
                                 batch.astype(jnp.int32).reshape(N, 1))
    return (node_repr, graph_repr)


# speedup vs baseline: 22.2011x; 1.0588x over previous
import functools

import jax
import jax.numpy as jnp
from jax import lax
from jax.experimental import pallas as pl
from jax.experimental.pallas import tpu as pltpu
from jax.experimental.pallas import tpu_sc as plsc

N = 10000
E = 320000
G = 64
NC = 2
NS = 16
CHUNK = 128
IB = 2048
EP = 327680
PAD = EP - E
EPT_FS = EP // NS
EPT_ES = EP // (NC * NS)
NP = 10240
RPT = NP // NS
ICH = 128
TRASH = NP - N
NDEG = 10752
BLK = 1000
GRID = N // BLK

_MESH = dict(core_axis_name="c", subcore_axis_name="s")


def _make_sc_scatter(edge_split):
    @functools.partial(
        pl.kernel,
        out_type=jax.ShapeDtypeStruct((2 * NP, 128), jnp.float32),
        mesh=plsc.VectorSubcoreMesh(**_MESH),
        scratch_types=[
            pltpu.VMEM((IB,), jnp.int32),
            pltpu.VMEM((IB,), jnp.int32),
            pltpu.VMEM((CHUNK, 128), jnp.float32),
            pltpu.VMEM((CHUNK, 128), jnp.float32),
            pltpu.VMEM_SHARED((NP, 128), jnp.float32),
            pltpu.SemaphoreType.DMA,
            pltpu.SemaphoreType.DMA,
            pltpu.SemaphoreType.DMA,
            pltpu.SemaphoreType.DMA,
        ],
    )
    def scat(y_hbm, src_hbm, dst_hbm, out_hbm, sidx, didx,
             rows0, rows1, acc, gsem0, gsem1, ssem0, ssem1):
        c = lax.axis_index("c")
        s = lax.axis_index("s")
        ybase = 0 if edge_split else c * NP
        for k in range(RPT // ICH):
            pltpu.sync_copy(y_hbm.at[pl.ds(ybase + s * RPT + k * ICH, ICH)],
                            rows0.at[pl.ds(0, ICH)])
            pltpu.sync_copy(rows0.at[pl.ds(0, ICH)],
                            acc.at[pl.ds(s * RPT + k * ICH, ICH)])
        plsc.subcore_barrier()

        if edge_split:
            nblk = EPT_ES // IB
            ebase = (c * NS + s) * EPT_ES
            sbase = ebase
        else:
            nblk = EPT_FS // IB
            ebase = s * EPT_FS
            sbase = c * EP + ebase

        rows = (rows0, rows1)
        gsem = (gsem0, gsem1)
        ssem = (ssem0, ssem1)
        nk = IB // CHUNK

        def blk(t, carry):
            pltpu.sync_copy(src_hbm.at[pl.ds(sbase + t * IB, IB)], sidx)
            pltpu.sync_copy(dst_hbm.at[pl.ds(ebase + t * IB, IB)], didx)
            pltpu.async_copy(
                y_hbm.at[sidx.at[pl.ds(0, CHUNK)]], rows[0], gsem[0])
            for k in range(nk):
                b = k % 2
                o = 1 - b
                gk = y_hbm.at[sidx.at[pl.ds(k * CHUNK, CHUNK)]]
                pltpu.make_async_copy(gk, rows[b], gsem[b]).wait()
                pltpu.async_copy(rows[b],
                                 acc.at[didx.at[pl.ds(k * CHUNK, CHUNK)]],
                                 ssem[b], add=True)
                if k >= 1:
                    pltpu.make_async_copy(
                        rows[o], acc.at[didx.at[pl.ds((k - 1) * CHUNK, CHUNK)]],
                        ssem[o]).wait()
                if k + 1 < nk:
                    pltpu.async_copy(
                        y_hbm.at[sidx.at[pl.ds((k + 1) * CHUNK, CHUNK)]],
                        rows[o], gsem[o])
            last = nk - 1
            pltpu.make_async_copy(
                rows[last % 2], acc.at[didx.at[pl.ds(last * CHUNK, CHUNK)]],
                ssem[last % 2]).wait()
            return carry

        lax.fori_loop(0, nblk, blk, 0)
        plsc.subcore_barrier()
        for k in range(RPT // ICH):
            pltpu.sync_copy(acc.at[pl.ds(s * RPT + k * ICH, ICH)],
                            rows0.at[pl.ds(0, ICH)])
            pltpu.sync_copy(rows0.at[pl.ds(0, ICH)],
                            out_hbm.at[pl.ds(c * NP + s * RPT + k * ICH, ICH)])

    return scat


_sc_scatter_es = _make_sc_scatter(True)
_sc_scatter_fs = _make_sc_scatter(False)


def _make_sc_deg():
    zlen = NDEG // NS
    CD = 1024

    @functools.partial(
        pl.kernel,
        out_type=jax.ShapeDtypeStruct((NC * NDEG,), jnp.float32),
        mesh=plsc.VectorSubcoreMesh(**_MESH),
        scratch_types=[
            pltpu.VMEM((CD,), jnp.int32),
            pltpu.VMEM((CD,), jnp.int32),
            pltpu.VMEM((CD,), jnp.float32),
            pltpu.VMEM((zlen,), jnp.float32),
            pltpu.VMEM_SHARED((NDEG,), jnp.float32),
            pltpu.SemaphoreType.DMA,
            pltpu.SemaphoreType.DMA,
        ],
    )
    def degk(dst_hbm, out_hbm, didx0, didx1, ones_v, zbuf, acc, isem0, isem1):
        c = lax.axis_index("c")
        s = lax.axis_index("s")
        for i in range(CD // 16):
            ones_v[pl.ds(16 * i, 16)] = jnp.ones((16,), jnp.float32)
        for i in range(zlen // 16):
            zbuf[pl.ds(16 * i, 16)] = jnp.zeros((16,), jnp.float32)
        pltpu.sync_copy(zbuf, acc.at[pl.ds(s * zlen, zlen)])
        plsc.subcore_barrier()

        ebase = (c * NS + s) * EPT_ES
        didx = (didx0, didx1)
        isem = (isem0, isem1)
        nch = EPT_ES // CD
        pltpu.async_copy(dst_hbm.at[pl.ds(ebase, CD)], didx0, isem0)

        def body(jj, carry):
            for b in range(2):
                j = jj * 2 + b
                pltpu.make_async_copy(dst_hbm.at[pl.ds(0, CD)],
                                      didx[b], isem[b]).wait()
                pltpu.async_copy(
                    dst_hbm.at[pl.ds(ebase + (j + 1) * CD, CD)],
                    didx[1 - b], isem[1 - b])
                pltpu.sync_copy(ones_v, acc.at[didx[b]], add=True)
            return carry

        lax.fori_loop(0, (nch - 2) // 2, body, 0)
        pltpu.make_async_copy(dst_hbm.at[pl.ds(0, CD)], didx0, isem0).wait()
        pltpu.async_copy(dst_hbm.at[pl.ds(ebase + (nch - 1) * CD, CD)],
                         didx1, isem1)
        pltpu.sync_copy(ones_v, acc.at[didx0], add=True)
        pltpu.make_async_copy(dst_hbm.at[pl.ds(0, CD)], didx1, isem1).wait()
        pltpu.sync_copy(ones_v, acc.at[didx1], add=True)
        plsc.subcore_barrier()
        pltpu.sync_copy(acc.at[pl.ds(s * zlen, zlen)], zbuf)
        pltpu.sync_copy(zbuf, out_hbm.at[pl.ds(c * NDEG + s * zlen, zlen)])

    return degk


_sc_deg = _make_sc_deg()


def _tc1_body(deg_ref, x_ref, dinv_ref, y0_ref):
    d = deg_ref[0] + deg_ref[1]
    dinv = lax.rsqrt(d + 1.0)
    dinv_ref[...] = dinv
    y0_ref[...] = x_ref[...] * dinv


def _tc1(degs, x):
    return pl.pallas_call(
        _tc1_body,
        grid=(GRID,),
        in_specs=[
            pl.BlockSpec((2, BLK, 1), lambda i: (0, i, 0)),
            pl.BlockSpec((BLK, 128), lambda i: (i, 0)),
        ],
        out_specs=[
            pl.BlockSpec((BLK, 1), lambda i: (i, 0)),
            pl.BlockSpec((BLK, 128), lambda i: (i, 0)),
        ],
        out_shape=[
            jax.ShapeDtypeStruct((N, 1), jnp.float32),
            jax.ShapeDtypeStruct((NP, 128), jnp.float32),
        ],
    )(degs, x)


def _tc2_body(z_ref, y0_ref, dinv_ref, w1_ref, b1_ref, y1_ref):
    dinv = dinv_ref[...]
    a = (z_ref[0] + z_ref[1] - y0_ref[...]) * dinv
    h = jnp.dot(a, w1_ref[...], preferred_element_type=jnp.float32)
    h = jnp.maximum(h + b1_ref[...], 0.0)
    y = h * dinv
    y1_ref[0] = y[:, :128]
    y1_ref[1] = y[:, 128:]


def _tc2(z1, y0, dinv, W1, b1):
    return pl.pallas_call(
        _tc2_body,
        grid=(GRID,),
        in_specs=[
            pl.BlockSpec((2, BLK, 128), lambda i: (0, i, 0)),
            pl.BlockSpec((BLK, 128), lambda i: (i, 0)),
            pl.BlockSpec((BLK, 1), lambda i: (i, 0)),
            pl.BlockSpec((128, 256), lambda i: (0, 0)),
            pl.BlockSpec((1, 256), lambda i: (0, 0)),
        ],
        out_specs=pl.BlockSpec((2, BLK, 128), lambda i: (0, i, 0)),
        out_shape=jax.ShapeDtypeStruct((2, NP, 128), jnp.float32),
    )(z1, y0, dinv, W1, b1)


def _tc3_body(z_ref, dinv_ref, w2_ref, b2_ref, w3_ref, y3_ref):
    dinv = dinv_ref[...]
    a = jnp.concatenate([z_ref[0], z_ref[1]], axis=1) * dinv
    h = jnp.dot(a, w2_ref[...], preferred_element_type=jnp.float32)
    h = jnp.maximum(h + b2_ref[...], 0.0)
    y3_ref[...] = jnp.dot(h, w3_ref[...],
                          preferred_element_type=jnp.float32) * dinv


def _tc3(z2, dinv, W2, b2, W3):
    return pl.pallas_call(
        _tc3_body,
        grid=(GRID,),
        in_specs=[
            pl.BlockSpec((2, BLK, 128), lambda i: (0, i, 0)),
            pl.BlockSpec((BLK, 1), lambda i: (i, 0)),
            pl.BlockSpec((256, 256), lambda i: (0, 0)),
            pl.BlockSpec((1, 256), lambda i: (0, 0)),
            pl.BlockSpec((256, 128), lambda i: (0, 0)),
        ],
        out_specs=pl.BlockSpec((BLK, 128), lambda i: (i, 0)),
        out_shape=jax.ShapeDtypeStruct((NP, 128), jnp.float32),
    )(z2, dinv, W2, b2, W3)


def _tc4_body(z_ref, y3_ref, dinv_ref, b3_ref, batch_ref, node_ref, graph_ref,
              pooled_acc, counts_acc):
    i = pl.program_id(0)
    node = ((z_ref[0] + z_ref[1] - y3_ref[...]) * dinv_ref[...]
            + b3_ref[...])
    node_ref[...] = node
    onehot = (batch_ref[...] ==
              lax.broadcasted_iota(jnp.int32, (BLK, G), 1)).astype(jnp.float32)
    pooled = lax.dot_general(onehot, node, (((0,), (0,)), ((), ())),
                             preferred_element_type=jnp.float32)
    cnt = lax.dot_general(onehot, jnp.ones((BLK, 1), jnp.float32),
                          (((0,), (0,)), ((), ())),
                          preferred_element_type=jnp.float32)

    @pl.when(i == 0)
    def _():
        pooled_acc[...] = jnp.zeros_like(pooled_acc)
        counts_acc[...] = jnp.zeros_like(counts_acc)

    pooled_acc[...] += pooled
    counts_acc[...] += cnt

    @pl.when(i == pl.num_programs(0) - 1)
    def _():
        graph_ref[...] = pooled_acc[...] / jnp.maximum(counts_acc[...], 1.0)


def _tc4(z3, y3, dinv, b3, batch2):
    return pl.pallas_call(
        _tc4_body,
        grid=(GRID,),
        in_specs=[
            pl.BlockSpec((2, BLK, 128), lambda i: (0, i, 0)),
            pl.BlockSpec((BLK, 128), lambda i: (i, 0)),
            pl.BlockSpec((BLK, 1), lambda i: (i, 0)),
            pl.BlockSpec((1, 128), lambda i: (0, 0)),
            pl.BlockSpec((BLK, 1), lambda i: (i, 0)),
        ],
        out_specs=[
            pl.BlockSpec((BLK, 128), lambda i: (i, 0)),
            pl.BlockSpec((G, 128), lambda i: (0, 0)),
        ],
        out_shape=[
            jax.ShapeDtypeStruct((N, 128), jnp.float32),
            jax.ShapeDtypeStruct((G, 128), jnp.float32),
        ],
        scratch_shapes=[
            pltpu.VMEM((G, 128), jnp.float32),
            pltpu.VMEM((G, 1), jnp.float32),
        ],
    )(z3, y3, dinv, b3, batch2)


def kernel(x, edge_index, batch, W1, b1, W2, b2, W3, b3):
    src = edge_index[0].astype(jnp.int32)
    dst = edge_index[1].astype(jnp.int32)
    pad_i = jnp.arange(PAD, dtype=jnp.int32)
    src_p = jnp.concatenate([src, pad_i % N])
    dst_p = jnp.concatenate([dst, N + (pad_i % TRASH)])
    src_fs = jnp.concatenate([src_p, src_p + NP])

    deg_flat = _sc_deg(dst_p)
    degs = deg_flat.reshape(NC, NDEG)[:, :N].reshape(2, N, 1)

    dinv, y0 = _tc1(degs, x)
    z1 = _sc_scatter_es(y0, src_p, dst_p).reshape(2, NP, 128)
    y1 = _tc2(z1, y0, dinv, W1, b1.reshape(1, 256))
    z2 = _sc_scatter_fs(y1.reshape(2 * NP, 128), src_fs, dst_p)
    y3 = _tc3(z2.reshape(2, NP, 128), dinv, W2, b2.reshape(1, 256), W3)
    z3 = _sc_scatter_es(y3, src_p, dst_p).reshape(2, NP, 128)
    node_repr, graph_repr = _tc4(z3, y3, dinv, b3.reshape(1, 128),
                                 batch.astype(jnp.int32).reshape(N, 1))
    return (node_repr, graph_repr)


# speedup vs baseline: 23.3706x; 1.0527x over previous
import functools

import jax
import jax.numpy as jnp
from jax import lax
from jax.experimental import pallas as pl
from jax.experimental.pallas import tpu as pltpu
from jax.experimental.pallas import tpu_sc as plsc

N = 10000
E = 320000
G = 64
NC = 2
NS = 16
CHUNK = 128
EP = 327680
PAD = EP - E
EPT_FS = EP // NS
EPT_ES = EP // (NC * NS)
NP = 10240
RPT = NP // NS
ICH = 128
TRASH = NP - N
NDEG = 10752
BLK = 1000
GRID = N // BLK

_MESH = dict(core_axis_name="c", subcore_axis_name="s")


def _make_sc_scatter(edge_split):
    IBK = 1024 if edge_split else 2048
    nk = IBK // CHUNK
    npairs = (EPT_ES if edge_split else EPT_FS) // IBK // 2

    @functools.partial(
        pl.kernel,
        out_type=jax.ShapeDtypeStruct((2 * NP, 128), jnp.float32),
        mesh=plsc.VectorSubcoreMesh(**_MESH),
        scratch_types=[
            pltpu.VMEM((IBK,), jnp.int32),
            pltpu.VMEM((IBK,), jnp.int32),
            pltpu.VMEM((IBK,), jnp.int32),
            pltpu.VMEM((IBK,), jnp.int32),
            pltpu.VMEM((CHUNK, 128), jnp.float32),
            pltpu.VMEM((CHUNK, 128), jnp.float32),
            pltpu.VMEM_SHARED((NP, 128), jnp.float32),
            pltpu.SemaphoreType.DMA,
            pltpu.SemaphoreType.DMA,
            pltpu.SemaphoreType.DMA,
            pltpu.SemaphoreType.DMA,
            pltpu.SemaphoreType.DMA,
            pltpu.SemaphoreType.DMA,
        ],
    )
    def scat(y_hbm, src_hbm, dst_hbm, out_hbm, sidxA, didxA, sidxB, didxB,
             rows0, rows1, acc, gsem0, gsem1, ssem0, ssem1, isemA, isemB):
        c = lax.axis_index("c")
        s = lax.axis_index("s")
        ybase = 0 if edge_split else c * NP
        for k in range(RPT // ICH):
            pltpu.sync_copy(y_hbm.at[pl.ds(ybase + s * RPT + k * ICH, ICH)],
                            rows0.at[pl.ds(0, ICH)])
            pltpu.sync_copy(rows0.at[pl.ds(0, ICH)],
                            acc.at[pl.ds(s * RPT + k * ICH, ICH)])
        plsc.subcore_barrier()

        if edge_split:
            ebase = (c * NS + s) * EPT_ES
            sbase = ebase
        else:
            ebase = s * EPT_FS
            sbase = c * EP + ebase

        rows = (rows0, rows1)
        gsem = (gsem0, gsem1)
        ssem = (ssem0, ssem1)
        sidx = (sidxA, sidxB)
        didx = (didxA, didxB)
        isem = (isemA, isemB)

        def wait_idx(iset):
            pltpu.make_async_copy(src_hbm.at[pl.ds(0, IBK)],
                                  sidx[iset], isem[iset]).wait()
            pltpu.make_async_copy(dst_hbm.at[pl.ds(0, IBK)],
                                  didx[iset], isem[iset]).wait()

        pltpu.sync_copy(src_hbm.at[pl.ds(sbase, IBK)], sidxA)
        pltpu.sync_copy(dst_hbm.at[pl.ds(ebase, IBK)], didxA)
        pltpu.async_copy(y_hbm.at[sidxA.at[pl.ds(0, CHUNK)]], rows0, gsem0)
        pltpu.async_copy(src_hbm.at[pl.ds(sbase + IBK, IBK)], sidxB, isemB)
        pltpu.async_copy(dst_hbm.at[pl.ds(ebase + IBK, IBK)], didxB, isemB)

        def pair(t2, carry):
            pbase_e = ebase + t2 * (2 * IBK)
            pbase_s = sbase + t2 * (2 * IBK)
            not_last = t2 < npairs - 1
            for k in range(2 * nk):
                b = k % 2
                o = 1 - b
                iset = 0 if k < nk else 1
                koff = (k - iset * nk) * CHUNK
                si = sidx[iset].at[pl.ds(koff, CHUNK)]
                di = didx[iset].at[pl.ds(koff, CHUNK)]
                pltpu.make_async_copy(y_hbm.at[si], rows[b], gsem[b]).wait()
                pltpu.async_copy(rows[b], acc.at[di], ssem[b], add=True)
                if k >= 1:
                    kp = k - 1
                    ipv = 0 if kp < nk else 1
                    dpv = didx[ipv].at[pl.ds((kp - ipv * nk) * CHUNK, CHUNK)]
                    pltpu.make_async_copy(rows[o], acc.at[dpv],
                                          ssem[o]).wait()
                if k == nk:
                    @pl.when(not_last)
                    def _():
                        pltpu.async_copy(
                            src_hbm.at[pl.ds(pbase_s + 2 * IBK, IBK)],
                            sidx[0], isem[0])
                        pltpu.async_copy(
                            dst_hbm.at[pl.ds(pbase_e + 2 * IBK, IBK)],
                            didx[0], isem[0])
                if k + 1 < 2 * nk:
                    kn = k + 1
                    inx = 0 if kn < nk else 1
                    if kn == nk:
                        wait_idx(1)
                    sn = sidx[inx].at[pl.ds((kn - inx * nk) * CHUNK, CHUNK)]
                    pltpu.async_copy(y_hbm.at[sn], rows[o], gsem[o])
                else:
                    @pl.when(not_last)
                    def _():
                        wait_idx(0)
                        pltpu.async_copy(y_hbm.at[sidx[0].at[pl.ds(0, CHUNK)]],
                                         rows[o], gsem[o])
            dlast = didx[1].at[pl.ds((nk - 1) * CHUNK, CHUNK)]
            pltpu.make_async_copy(rows[1], acc.at[dlast], ssem[1]).wait()

            @pl.when(not_last)
            def _():
                pltpu.async_copy(src_hbm.at[pl.ds(pbase_s + 3 * IBK, IBK)],
                                 sidx[1], isem[1])
                pltpu.async_copy(dst_hbm.at[pl.ds(pbase_e + 3 * IBK, IBK)],
                                 didx[1], isem[1])
            return carry

        lax.fori_loop(0, npairs, pair, 0)
        plsc.subcore_barrier()
        for k in range(RPT // ICH):
            pltpu.sync_copy(acc.at[pl.ds(s * RPT + k * ICH, ICH)],
                            rows0.at[pl.ds(0, ICH)])
            pltpu.sync_copy(rows0.at[pl.ds(0, ICH)],
                            out_hbm.at[pl.ds(c * NP + s * RPT + k * ICH, ICH)])

    return scat


_sc_scatter_es = _make_sc_scatter(True)
_sc_scatter_fs = _make_sc_scatter(False)


def _make_sc_deg():
    zlen = NDEG // NS
    CD = 1024

    @functools.partial(
        pl.kernel,
        out_type=jax.ShapeDtypeStruct((NC * NDEG,), jnp.float32),
        mesh=plsc.VectorSubcoreMesh(**_MESH),
        scratch_types=[
            pltpu.VMEM((CD,), jnp.int32),
            pltpu.VMEM((CD,), jnp.int32),
            pltpu.VMEM((CD,), jnp.float32),
            pltpu.VMEM((zlen,), jnp.float32),
            pltpu.VMEM_SHARED((NDEG,), jnp.float32),
            pltpu.SemaphoreType.DMA,
            pltpu.SemaphoreType.DMA,
        ],
    )
    def degk(dst_hbm, out_hbm, didx0, didx1, ones_v, zbuf, acc, isem0, isem1):
        c = lax.axis_index("c")
        s = lax.axis_index("s")
        for i in range(CD // 16):
            ones_v[pl.ds(16 * i, 16)] = jnp.ones((16,), jnp.float32)
        for i in range(zlen // 16):
            zbuf[pl.ds(16 * i, 16)] = jnp.zeros((16,), jnp.float32)
        pltpu.sync_copy(zbuf, acc.at[pl.ds(s * zlen, zlen)])
        plsc.subcore_barrier()

        ebase = (c * NS + s) * EPT_ES
        didx = (didx0, didx1)
        isem = (isem0, isem1)
        nch = EPT_ES // CD
        pltpu.async_copy(dst_hbm.at[pl.ds(ebase, CD)], didx0, isem0)

        def body(jj, carry):
            for b in range(2):
                j = jj * 2 + b
                pltpu.make_async_copy(dst_hbm.at[pl.ds(0, CD)],
                                      didx[b], isem[b]).wait()
                pltpu.async_copy(
                    dst_hbm.at[pl.ds(ebase + (j + 1) * CD, CD)],
                    didx[1 - b], isem[1 - b])
                pltpu.sync_copy(ones_v, acc.at[didx[b]], add=True)
            return carry

        lax.fori_loop(0, (nch - 2) // 2, body, 0)
        pltpu.make_async_copy(dst_hbm.at[pl.ds(0, CD)], didx0, isem0).wait()
        pltpu.async_copy(dst_hbm.at[pl.ds(ebase + (nch - 1) * CD, CD)],
                         didx1, isem1)
        pltpu.sync_copy(ones_v, acc.at[didx0], add=True)
        pltpu.make_async_copy(dst_hbm.at[pl.ds(0, CD)], didx1, isem1).wait()
        pltpu.sync_copy(ones_v, acc.at[didx1], add=True)
        plsc.subcore_barrier()
        pltpu.sync_copy(acc.at[pl.ds(s * zlen, zlen)], zbuf)
        pltpu.sync_copy(zbuf, out_hbm.at[pl.ds(c * NDEG + s * zlen, zlen)])

    return degk


_sc_deg = _make_sc_deg()


def _tc1_body(deg_ref, x_ref, dinv_ref, y0_ref):
    d = deg_ref[0] + deg_ref[1]
    dinv = lax.rsqrt(d + 1.0)
    dinv_ref[...] = dinv
    y0_ref[...] = x_ref[...] * dinv


def _tc1(degs, x):
    return pl.pallas_call(
        _tc1_body,
        grid=(GRID,),
        in_specs=[
            pl.BlockSpec((2, BLK, 1), lambda i: (0, i, 0)),
            pl.BlockSpec((BLK, 128), lambda i: (i, 0)),
        ],
        out_specs=[
            pl.BlockSpec((BLK, 1), lambda i: (i, 0)),
            pl.BlockSpec((BLK, 128), lambda i: (i, 0)),
        ],
        out_shape=[
            jax.ShapeDtypeStruct((N, 1), jnp.float32),
            jax.ShapeDtypeStruct((NP, 128), jnp.float32),
        ],
    )(degs, x)


def _tc2_body(z_ref, y0_ref, dinv_ref, w1_ref, b1_ref, y1_ref):
    dinv = dinv_ref[...]
    a = (z_ref[0] + z_ref[1] - y0_ref[...]) * dinv
    h = jnp.dot(a, w1_ref[...], preferred_element_type=jnp.float32)
    h = jnp.maximum(h + b1_ref[...], 0.0)
    y = h * dinv
    y1_ref[0] = y[:, :128]
    y1_ref[1] = y[:, 128:]


def _tc2(z1, y0, dinv, W1, b1):
    return pl.pallas_call(
        _tc2_body,
        grid=(GRID,),
        in_specs=[
            pl.BlockSpec((2, BLK, 128), lambda i: (0, i, 0)),
            pl.BlockSpec((BLK, 128), lambda i: (i, 0)),
            pl.BlockSpec((BLK, 1), lambda i: (i, 0)),
            pl.BlockSpec((128, 256), lambda i: (0, 0)),
            pl.BlockSpec((1, 256), lambda i: (0, 0)),
        ],
        out_specs=pl.BlockSpec((2, BLK, 128), lambda i: (0, i, 0)),
        out_shape=jax.ShapeDtypeStruct((2, NP, 128), jnp.float32),
    )(z1, y0, dinv, W1, b1)


def _tc3_body(z_ref, dinv_ref, w2_ref, b2_ref, w3_ref, y3_ref):
    dinv = dinv_ref[...]
    a = jnp.concatenate([z_ref[0], z_ref[1]], axis=1) * dinv
    h = jnp.dot(a, w2_ref[...], preferred_element_type=jnp.float32)
    h = jnp.maximum(h + b2_ref[...], 0.0)
    y3_ref[...] = jnp.dot(h, w3_ref[...],
                          preferred_element_type=jnp.float32) * dinv


def _tc3(z2, dinv, W2, b2, W3):
    return pl.pallas_call(
        _tc3_body,
        grid=(GRID,),
        in_specs=[
            pl.BlockSpec((2, BLK, 128), lambda i: (0, i, 0)),
            pl.BlockSpec((BLK, 1), lambda i: (i, 0)),
            pl.BlockSpec((256, 256), lambda i: (0, 0)),
            pl.BlockSpec((1, 256), lambda i: (0, 0)),
            pl.BlockSpec((256, 128), lambda i: (0, 0)),
        ],
        out_specs=pl.BlockSpec((BLK, 128), lambda i: (i, 0)),
        out_shape=jax.ShapeDtypeStruct((NP, 128), jnp.float32),
    )(z2, dinv, W2, b2, W3)


def _tc4_body(z_ref, y3_ref, dinv_ref, b3_ref, batch_ref, node_ref, graph_ref,
              pooled_acc, counts_acc):
    i = pl.program_id(0)
    node = ((z_ref[0] + z_ref[1] - y3_ref[...]) * dinv_ref[...]
            + b3_ref[...])
    node_ref[...] = node
    onehot = (batch_ref[...] ==
              lax.broadcasted_iota(jnp.int32, (BLK, G), 1)).astype(jnp.float32)
    pooled = lax.dot_general(onehot, node, (((0,), (0,)), ((), ())),
                             preferred_element_type=jnp.float32)
    cnt = lax.dot_general(onehot, jnp.ones((BLK, 1), jnp.float32),
                          (((0,), (0,)), ((), ())),
                          preferred_element_type=jnp.float32)

    @pl.when(i == 0)
    def _():
        pooled_acc[...] = jnp.zeros_like(pooled_acc)
        counts_acc[...] = jnp.zeros_like(counts_acc)

    pooled_acc[...] += pooled
    counts_acc[...] += cnt

    @pl.when(i == pl.num_programs(0) - 1)
    def _():
        graph_ref[...] = pooled_acc[...] / jnp.maximum(counts_acc[...], 1.0)


def _tc4(z3, y3, dinv, b3, batch2):
    return pl.pallas_call(
        _tc4_body,
        grid=(GRID,),
        in_specs=[
            pl.BlockSpec((2, BLK, 128), lambda i: (0, i, 0)),
            pl.BlockSpec((BLK, 128), lambda i: (i, 0)),
            pl.BlockSpec((BLK, 1), lambda i: (i, 0)),
            pl.BlockSpec((1, 128), lambda i: (0, 0)),
            pl.BlockSpec((BLK, 1), lambda i: (i, 0)),
        ],
        out_specs=[
            pl.BlockSpec((BLK, 128), lambda i: (i, 0)),
            pl.BlockSpec((G, 128), lambda i: (0, 0)),
        ],
        out_shape=[
            jax.ShapeDtypeStruct((N, 128), jnp.float32),
            jax.ShapeDtypeStruct((G, 128), jnp.float32),
        ],
        scratch_shapes=[
            pltpu.VMEM((G, 128), jnp.float32),
            pltpu.VMEM((G, 1), jnp.float32),
        ],
    )(z3, y3, dinv, b3, batch2)


def kernel(x, edge_index, batch, W1, b1, W2, b2, W3, b3):
    src = edge_index[0].astype(jnp.int32)
    dst = edge_index[1].astype(jnp.int32)
    pad_i = jnp.arange(PAD, dtype=jnp.int32)
    src_p = jnp.concatenate([src, pad_i % N])
    dst_p = jnp.concatenate([dst, N + (pad_i % TRASH)])
    src_fs = jnp.concatenate([src_p, src_p + NP])

    deg_flat = _sc_deg(dst_p)
    degs = deg_flat.reshape(NC, NDEG)[:, :N].reshape(2, N, 1)

    dinv, y0 = _tc1(degs, x)
    z1 = _sc_scatter_es(y0, src_p, dst_p).reshape(2, NP, 128)
    y1 = _tc2(z1, y0, dinv, W1, b1.reshape(1, 256))
    z2 = _sc_scatter_fs(y1.reshape(2 * NP, 128), src_fs, dst_p)
    y3 = _tc3(z2.reshape(2, NP, 128), dinv, W2, b2.reshape(1, 256), W3)
    z3 = _sc_scatter_es(y3, src_p, dst_p).reshape(2, NP, 128)
    node_repr, graph_repr = _tc4(z3, y3, dinv, b3.reshape(1, 128),
                                 batch.astype(jnp.int32).reshape(N, 1))
    return (node_repr, graph_repr)


# speedup vs baseline: 23.7431x; 1.0159x over previous
import functools

import jax
import jax.numpy as jnp
from jax import lax
from jax.experimental import pallas as pl
from jax.experimental.pallas import tpu as pltpu
from jax.experimental.pallas import tpu_sc as plsc

N = 10000
E = 320000
G = 64
NC = 2
NS = 16
CHUNK = 128
EP = 327680
PAD = EP - E
EPT_FS = EP // NS
EPT_ES = EP // (NC * NS)
NP = 10240
RPT = NP // NS
ICH = 128
TRASH = NP - N
NDEG = 10752
BLK = 1000
GRID = N // BLK

_MESH = dict(core_axis_name="c", subcore_axis_name="s")


def _make_sc_scatter(edge_split):
    IBK = 1024 if edge_split else 2048
    nk = IBK // CHUNK
    npairs = (EPT_ES if edge_split else EPT_FS) // IBK // 2

    @functools.partial(
        pl.kernel,
        out_type=jax.ShapeDtypeStruct((2 * NP, 128), jnp.float32),
        mesh=plsc.VectorSubcoreMesh(**_MESH),
        scratch_types=[
            pltpu.VMEM((IBK,), jnp.int32),
            pltpu.VMEM((IBK,), jnp.int32),
            pltpu.VMEM((IBK,), jnp.int32),
            pltpu.VMEM((IBK,), jnp.int32),
            pltpu.VMEM((CHUNK, 128), jnp.float32),
            pltpu.VMEM((CHUNK, 128), jnp.float32),
            pltpu.VMEM_SHARED((NP, 128), jnp.float32),
            pltpu.SemaphoreType.DMA,
            pltpu.SemaphoreType.DMA,
            pltpu.SemaphoreType.DMA,
            pltpu.SemaphoreType.DMA,
            pltpu.SemaphoreType.DMA,
            pltpu.SemaphoreType.DMA,
        ],
    )
    def scat(y_hbm, src_hbm, dst_hbm, out_hbm, sidxA, didxA, sidxB, didxB,
             rows0, rows1, acc, gsem0, gsem1, ssem0, ssem1, isemA, isemB):
        c = lax.axis_index("c")
        s = lax.axis_index("s")
        ybase = 0 if edge_split else c * NP
        nic = RPT // ICH
        ibufs = (rows0, rows1)
        isems_i = (gsem0, gsem1)
        pltpu.async_copy(y_hbm.at[pl.ds(ybase + s * RPT, ICH)],
                         rows0.at[pl.ds(0, ICH)], gsem0)
        for k in range(nic):
            b = k % 2
            pltpu.make_async_copy(y_hbm.at[pl.ds(0, ICH)],
                                  ibufs[b].at[pl.ds(0, ICH)],
                                  isems_i[b]).wait()
            if k + 1 < nic:
                pltpu.async_copy(
                    y_hbm.at[pl.ds(ybase + s * RPT + (k + 1) * ICH, ICH)],
                    ibufs[1 - b].at[pl.ds(0, ICH)], isems_i[1 - b])
            pltpu.sync_copy(ibufs[b].at[pl.ds(0, ICH)],
                            acc.at[pl.ds(s * RPT + k * ICH, ICH)])
        plsc.subcore_barrier()

        if edge_split:
            ebase = (c * NS + s) * EPT_ES
            sbase = ebase
        else:
            ebase = s * EPT_FS
            sbase = c * EP + ebase

        rows = (rows0, rows1)
        gsem = (gsem0, gsem1)
        ssem = (ssem0, ssem1)
        sidx = (sidxA, sidxB)
        didx = (didxA, didxB)
        isem = (isemA, isemB)

        def wait_idx(iset):
            pltpu.make_async_copy(src_hbm.at[pl.ds(0, IBK)],
                                  sidx[iset], isem[iset]).wait()
            pltpu.make_async_copy(dst_hbm.at[pl.ds(0, IBK)],
                                  didx[iset], isem[iset]).wait()

        pltpu.sync_copy(src_hbm.at[pl.ds(sbase, IBK)], sidxA)
        pltpu.sync_copy(dst_hbm.at[pl.ds(ebase, IBK)], didxA)
        pltpu.async_copy(y_hbm.at[sidxA.at[pl.ds(0, CHUNK)]], rows0, gsem0)
        pltpu.async_copy(src_hbm.at[pl.ds(sbase + IBK, IBK)], sidxB, isemB)
        pltpu.async_copy(dst_hbm.at[pl.ds(ebase + IBK, IBK)], didxB, isemB)

        def pair(t2, carry):
            pbase_e = ebase + t2 * (2 * IBK)
            pbase_s = sbase + t2 * (2 * IBK)
            not_last = t2 < npairs - 1
            for k in range(2 * nk):
                b = k % 2
                o = 1 - b
                iset = 0 if k < nk else 1
                koff = (k - iset * nk) * CHUNK
                si = sidx[iset].at[pl.ds(koff, CHUNK)]
                di = didx[iset].at[pl.ds(koff, CHUNK)]
                pltpu.make_async_copy(y_hbm.at[si], rows[b], gsem[b]).wait()
                pltpu.async_copy(rows[b], acc.at[di], ssem[b], add=True)
                if k >= 1:
                    kp = k - 1
                    ipv = 0 if kp < nk else 1
                    dpv = didx[ipv].at[pl.ds((kp - ipv * nk) * CHUNK, CHUNK)]
                    pltpu.make_async_copy(rows[o], acc.at[dpv],
                                          ssem[o]).wait()
                if k == nk:
                    @pl.when(not_last)
                    def _():
                        pltpu.async_copy(
                            src_hbm.at[pl.ds(pbase_s + 2 * IBK, IBK)],
                            sidx[0], isem[0])
                        pltpu.async_copy(
                            dst_hbm.at[pl.ds(pbase_e + 2 * IBK, IBK)],
                            didx[0], isem[0])
                if k + 1 < 2 * nk:
                    kn = k + 1
                    inx = 0 if kn < nk else 1
                    if kn == nk:
                        wait_idx(1)
                    sn = sidx[inx].at[pl.ds((kn - inx * nk) * CHUNK, CHUNK)]
                    pltpu.async_copy(y_hbm.at[sn], rows[o], gsem[o])
                else:
                    @pl.when(not_last)
                    def _():
                        wait_idx(0)
                        pltpu.async_copy(y_hbm.at[sidx[0].at[pl.ds(0, CHUNK)]],
                                         rows[o], gsem[o])
            dlast = didx[1].at[pl.ds((nk - 1) * CHUNK, CHUNK)]
            pltpu.make_async_copy(rows[1], acc.at[dlast], ssem[1]).wait()

            @pl.when(not_last)
            def _():
                pltpu.async_copy(src_hbm.at[pl.ds(pbase_s + 3 * IBK, IBK)],
                                 sidx[1], isem[1])
                pltpu.async_copy(dst_hbm.at[pl.ds(pbase_e + 3 * IBK, IBK)],
                                 didx[1], isem[1])
            return carry

        lax.fori_loop(0, npairs, pair, 0)
        plsc.subcore_barrier()
        obase = c * NP + s * RPT
        pltpu.async_copy(acc.at[pl.ds(s * RPT, ICH)],
                         rows0.at[pl.ds(0, ICH)], gsem0)
        for k in range(nic):
            b = k % 2
            pltpu.make_async_copy(y_hbm.at[pl.ds(0, ICH)],
                                  ibufs[b].at[pl.ds(0, ICH)],
                                  isems_i[b]).wait()
            pltpu.async_copy(ibufs[b].at[pl.ds(0, ICH)],
                             out_hbm.at[pl.ds(obase + k * ICH, ICH)],
                             ssem[b])
            if k + 1 < nic:
                if k >= 1:
                    pltpu.make_async_copy(
                        ibufs[1 - b].at[pl.ds(0, ICH)],
                        out_hbm.at[pl.ds(obase, ICH)], ssem[1 - b]).wait()
                pltpu.async_copy(
                    acc.at[pl.ds(s * RPT + (k + 1) * ICH, ICH)],
                    ibufs[1 - b].at[pl.ds(0, ICH)], isems_i[1 - b])
        pltpu.make_async_copy(ibufs[(nic - 1) % 2].at[pl.ds(0, ICH)],
                              out_hbm.at[pl.ds(obase, ICH)],
                              ssem[(nic - 1) % 2]).wait()
        pltpu.make_async_copy(ibufs[nic % 2].at[pl.ds(0, ICH)],
                              out_hbm.at[pl.ds(obase, ICH)],
                              ssem[nic % 2]).wait()

    return scat


_sc_scatter_es = _make_sc_scatter(True)
_sc_scatter_fs = _make_sc_scatter(False)


def _make_sc_deg():
    zlen = NDEG // NS
    CD = 1024

    @functools.partial(
        pl.kernel,
        out_type=jax.ShapeDtypeStruct((NC * NDEG,), jnp.float32),
        mesh=plsc.VectorSubcoreMesh(**_MESH),
        scratch_types=[
            pltpu.VMEM((CD,), jnp.int32),
            pltpu.VMEM((CD,), jnp.int32),
            pltpu.VMEM((CD,), jnp.float32),
            pltpu.VMEM((zlen,), jnp.float32),
            pltpu.VMEM_SHARED((NDEG,), jnp.float32),
            pltpu.SemaphoreType.DMA,
            pltpu.SemaphoreType.DMA,
        ],
    )
    def degk(dst_hbm, out_hbm, didx0, didx1, ones_v, zbuf, acc, isem0, isem1):
        c = lax.axis_index("c")
        s = lax.axis_index("s")
        for i in range(CD // 16):
            ones_v[pl.ds(16 * i, 16)] = jnp.ones((16,), jnp.float32)
        for i in range(zlen // 16):
            zbuf[pl.ds(16 * i, 16)] = jnp.zeros((16,), jnp.float32)
        pltpu.sync_copy(zbuf, acc.at[pl.ds(s * zlen, zlen)])
        plsc.subcore_barrier()

        ebase = (c * NS + s) * EPT_ES
        didx = (didx0, didx1)
        isem = (isem0, isem1)
        nch = EPT_ES // CD
        pltpu.async_copy(dst_hbm.at[pl.ds(ebase, CD)], didx0, isem0)

        def body(jj, carry):
            for b in range(2):
                j = jj * 2 + b
                pltpu.make_async_copy(dst_hbm.at[pl.ds(0, CD)],
                                      didx[b], isem[b]).wait()
                pltpu.async_copy(
                    dst_hbm.at[pl.ds(ebase + (j + 1) * CD, CD)],
                    didx[1 - b], isem[1 - b])
                pltpu.sync_copy(ones_v, acc.at[didx[b]], add=True)
            return carry

        lax.fori_loop(0, (nch - 2) // 2, body, 0)
        pltpu.make_async_copy(dst_hbm.at[pl.ds(0, CD)], didx0, isem0).wait()
        pltpu.async_copy(dst_hbm.at[pl.ds(ebase + (nch - 1) * CD, CD)],
                         didx1, isem1)
        pltpu.sync_copy(ones_v, acc.at[didx0], add=True)
        pltpu.make_async_copy(dst_hbm.at[pl.ds(0, CD)], didx1, isem1).wait()
        pltpu.sync_copy(ones_v, acc.at[didx1], add=True)
        plsc.subcore_barrier()
        pltpu.sync_copy(acc.at[pl.ds(s * zlen, zlen)], zbuf)
        pltpu.sync_copy(zbuf, out_hbm.at[pl.ds(c * NDEG + s * zlen, zlen)])

    return degk


_sc_deg = _make_sc_deg()


def _tc1_body(deg_ref, x_ref, dinv_ref, y0_ref):
    d = deg_ref[0] + deg_ref[1]
    dinv = lax.rsqrt(d + 1.0)
    dinv_ref[...] = dinv
    y0_ref[...] = x_ref[...] * dinv


def _tc1(degs, x):
    return pl.pallas_call(
        _tc1_body,
        grid=(GRID,),
        in_specs=[
            pl.BlockSpec((2, BLK, 1), lambda i: (0, i, 0)),
            pl.BlockSpec((BLK, 128), lambda i: (i, 0)),
        ],
        out_specs=[
            pl.BlockSpec((BLK, 1), lambda i: (i, 0)),
            pl.BlockSpec((BLK, 128), lambda i: (i, 0)),
        ],
        out_shape=[
            jax.ShapeDtypeStruct((N, 1), jnp.float32),
            jax.ShapeDtypeStruct((NP, 128), jnp.float32),
        ],
    )(degs, x)


def _tc2_body(z_ref, y0_ref, dinv_ref, w1_ref, b1_ref, y1_ref):
    dinv = dinv_ref[...]
    a = (z_ref[0] + z_ref[1] - y0_ref[...]) * dinv
    h = jnp.dot(a, w1_ref[...], preferred_element_type=jnp.float32)
    h = jnp.maximum(h + b1_ref[...], 0.0)
    y = h * dinv
    y1_ref[0] = y[:, :128]
    y1_ref[1] = y[:, 128:]


def _tc2(z1, y0, dinv, W1, b1):
    return pl.pallas_call(
        _tc2_body,
        grid=(GRID,),
        in_specs=[
            pl.BlockSpec((2, BLK, 128), lambda i: (0, i, 0)),
            pl.BlockSpec((BLK, 128), lambda i: (i, 0)),
            pl.BlockSpec((BLK, 1), lambda i: (i, 0)),
            pl.BlockSpec((128, 256), lambda i: (0, 0)),
            pl.BlockSpec((1, 256), lambda i: (0, 0)),
        ],
        out_specs=pl.BlockSpec((2, BLK, 128), lambda i: (0, i, 0)),
        out_shape=jax.ShapeDtypeStruct((2, NP, 128), jnp.float32),
    )(z1, y0, dinv, W1, b1)


def _tc3_body(z_ref, dinv_ref, w2_ref, b2_ref, w3_ref, y3_ref):
    dinv = dinv_ref[...]
    a = jnp.concatenate([z_ref[0], z_ref[1]], axis=1) * dinv
    h = jnp.dot(a, w2_ref[...], preferred_element_type=jnp.float32)
    h = jnp.maximum(h + b2_ref[...], 0.0)
    y3_ref[...] = jnp.dot(h, w3_ref[...],
                          preferred_element_type=jnp.float32) * dinv


def _tc3(z2, dinv, W2, b2, W3):
    return pl.pallas_call(
        _tc3_body,
        grid=(GRID,),
        in_specs=[
            pl.BlockSpec((2, BLK, 128), lambda i: (0, i, 0)),
            pl.BlockSpec((BLK, 1), lambda i: (i, 0)),
            pl.BlockSpec((256, 256), lambda i: (0, 0)),
            pl.BlockSpec((1, 256), lambda i: (0, 0)),
            pl.BlockSpec((256, 128), lambda i: (0, 0)),
        ],
        out_specs=pl.BlockSpec((BLK, 128), lambda i: (i, 0)),
        out_shape=jax.ShapeDtypeStruct((NP, 128), jnp.float32),
    )(z2, dinv, W2, b2, W3)


def _tc4_body(z_ref, y3_ref, dinv_ref, b3_ref, batch_ref, node_ref, graph_ref,
              pooled_acc, counts_acc):
    i = pl.program_id(0)
    node = ((z_ref[0] + z_ref[1] - y3_ref[...]) * dinv_ref[...]
            + b3_ref[...])
    node_ref[...] = node
    onehot = (batch_ref[...] ==
              lax.broadcasted_iota(jnp.int32, (BLK, G), 1)).astype(jnp.float32)
    pooled = lax.dot_general(onehot, node, (((0,), (0,)), ((), ())),
                             preferred_element_type=jnp.float32)
    cnt = lax.dot_general(onehot, jnp.ones((BLK, 1), jnp.float32),
                          (((0,), (0,)), ((), ())),
                          preferred_element_type=jnp.float32)

    @pl.when(i == 0)
    def _():
        pooled_acc[...] = jnp.zeros_like(pooled_acc)
        counts_acc[...] = jnp.zeros_like(counts_acc)

    pooled_acc[...] += pooled
    counts_acc[...] += cnt

    @pl.when(i == pl.num_programs(0) - 1)
    def _():
        graph_ref[...] = pooled_acc[...] / jnp.maximum(counts_acc[...], 1.0)


def _tc4(z3, y3, dinv, b3, batch2):
    return pl.pallas_call(
        _tc4_body,
        grid=(GRID,),
        in_specs=[
            pl.BlockSpec((2, BLK, 128), lambda i: (0, i, 0)),
            pl.BlockSpec((BLK, 128), lambda i: (i, 0)),
            pl.BlockSpec((BLK, 1), lambda i: (i, 0)),
            pl.BlockSpec((1, 128), lambda i: (0, 0)),
            pl.BlockSpec((BLK, 1), lambda i: (i, 0)),
        ],
        out_specs=[
            pl.BlockSpec((BLK, 128), lambda i: (i, 0)),
            pl.BlockSpec((G, 128), lambda i: (0, 0)),
        ],
        out_shape=[
            jax.ShapeDtypeStruct((N, 128), jnp.float32),
            jax.ShapeDtypeStruct((G, 128), jnp.float32),
        ],
        scratch_shapes=[
            pltpu.VMEM((G, 128), jnp.float32),
            pltpu.VMEM((G, 1), jnp.float32),
        ],
    )(z3, y3, dinv, b3, batch2)


def kernel(x, edge_index, batch, W1, b1, W2, b2, W3, b3):
    src = edge_index[0].astype(jnp.int32)
    dst = edge_index[1].astype(jnp.int32)
    pad_i = jnp.arange(PAD, dtype=jnp.int32)
    src_p = jnp.concatenate([src, pad_i % N])
    dst_p = jnp.concatenate([dst, N + (pad_i % TRASH)])
    src_fs = jnp.concatenate([src_p, src_p + NP])

    deg_flat = _sc_deg(dst_p)
    degs = deg_flat.reshape(NC, NDEG)[:, :N].reshape(2, N, 1)

    dinv, y0 = _tc1(degs, x)
    z1 = _sc_scatter_es(y0, src_p, dst_p).reshape(2, NP, 128)
    y1 = _tc2(z1, y0, dinv, W1, b1.reshape(1, 256))
    z2 = _sc_scatter_fs(y1.reshape(2 * NP, 128), src_fs, dst_p)
    y3 = _tc3(z2.reshape(2, NP, 128), dinv, W2, b2.reshape(1, 256), W3)
    z3 = _sc_scatter_es(y3, src_p, dst_p).reshape(2, NP, 128)
    node_repr, graph_repr = _tc4(z3, y3, dinv, b3.reshape(1, 128),
                                 batch.astype(jnp.int32).reshape(N, 1))
    return (node_repr, graph_repr)
